# baseline (speedup 1.0000x reference)
import functools

import jax
import jax.numpy as jnp
from jax import lax
from jax.experimental import pallas as pl
from jax.experimental.pallas import tpu as pltpu
from jax.experimental.pallas import tpu_sc as plsc

N = 10000
E = 160000
D_IN = 32
D_EDGE = 16
HID = 32

ET = 640
EGRID = E // ET

NC = 2
NS = 16
NW = NC * NS
EPW = E // NW
CHUNK = 128
NFULL = EPW // CHUNK
TAIL = EPW - NFULL * CHUNK
NPT = N // NS


def _edges_body(ea_ref, xj_ref, A1_ref, b1_ref, A2m_ref, B2_ref, R_ref, T_ref, out_ref, *, ones_col):
    ea = ea_ref[...]
    xj = xj_ref[...]
    h = jnp.maximum(
        jnp.dot(ea, A1_ref[...], preferred_element_type=jnp.float32) + b1_ref[...],
        0.0)
    hb = h.astype(jnp.bfloat16)
    h_rep = jnp.dot(hb, R_ref[...].astype(jnp.bfloat16),
                    preferred_element_type=jnp.float32).astype(jnp.bfloat16)
    x_rep = jnp.dot(xj.astype(jnp.bfloat16), T_ref[...].astype(jnp.bfloat16),
                    preferred_element_type=jnp.float32).astype(jnp.bfloat16)
    z = h_rep * x_rep
    msg = jnp.dot(z, A2m_ref[...].astype(jnp.bfloat16),
                  preferred_element_type=jnp.float32)
    msg = msg + jnp.dot(xj, B2_ref[...], preferred_element_type=jnp.float32)
    if ones_col:
        pad = jnp.concatenate(
            [msg, jnp.ones((ET, 1), jnp.float32), jnp.zeros((ET, 15), jnp.float32)],
            axis=1)
        out_ref[...] = pad
    else:
        out_ref[...] = msg


def _edge_messages(edge_attr, xj, A1, b1, A2m, B2, R, T, ones_col):
    width = HID + 16 if ones_col else HID
    return pl.pallas_call(
        functools.partial(_edges_body, ones_col=ones_col),
        grid=(EGRID,),
        in_specs=[
            pl.BlockSpec((ET, D_EDGE), lambda i: (i, 0)),
            pl.BlockSpec((ET, D_IN), lambda i: (i, 0)),
            pl.BlockSpec((D_EDGE, HID), lambda i: (0, 0)),
            pl.BlockSpec((1, HID), lambda i: (0, 0)),
            pl.BlockSpec((HID * D_IN, HID), lambda i: (0, 0)),
            pl.BlockSpec((D_IN, HID), lambda i: (0, 0)),
            pl.BlockSpec((HID, HID * D_IN), lambda i: (0, 0)),
            pl.BlockSpec((D_IN, HID * D_IN), lambda i: (0, 0)),
        ],
        out_specs=pl.BlockSpec((ET, width), lambda i: (i, 0)),
        out_shape=jax.ShapeDtypeStruct((E, width), jnp.float32),
    )(edge_attr, xj, A1, b1, A2m, B2, R, T)


def _node1_body(part_ref, x_ref, root_ref, bias_ref, h_ref, cnt_ref):
    p = part_ref[0] + part_ref[1]
    cnt = p[:, HID:HID + 1]
    denom = jnp.maximum(cnt, 1.0)
    agg = p[:, :HID] / denom
    r = jnp.dot(x_ref[...], root_ref[...], preferred_element_type=jnp.float32)
    h_ref[...] = jnp.maximum(agg + r + bias_ref[...], 0.0)
    cnt_ref[...] = cnt


def _node2_body(part_ref, cnt_ref, h_ref, root_ref, bias_ref, Wo_ref, bo_ref, out_ref):
    p = part_ref[0] + part_ref[1]
    denom = jnp.maximum(cnt_ref[...], 1.0)
    agg = p / denom
    r = jnp.dot(h_ref[...], root_ref[...], preferred_element_type=jnp.float32)
    h2 = jnp.maximum(agg + r + bias_ref[...], 0.0)
    logit = jnp.dot(h2, Wo_ref[...], preferred_element_type=jnp.float32) + bo_ref[...]
    out_ref[...] = jax.nn.sigmoid(logit)


def _node1(partial, x, root1, bias1):
    return pl.pallas_call(
        _node1_body,
        out_shape=(jax.ShapeDtypeStruct((N, HID), jnp.float32),
                   jax.ShapeDtypeStruct((N, 1), jnp.float32)),
    )(partial, x, root1, bias1)


def _node2(partial, cnt, h, root2, bias2, Wo, bo):
    return pl.pallas_call(
        _node2_body,
        out_shape=jax.ShapeDtypeStruct((N, 1), jnp.float32),
    )(partial, cnt, h, root2, bias2, Wo, bo)


def _sc_gather(table, idx):
    mesh = plsc.VectorSubcoreMesh(core_axis_name="c", subcore_axis_name="s")

    @functools.partial(
        pl.kernel, mesh=mesh,
        compiler_params=pltpu.CompilerParams(use_tc_tiling_on_sc=False),
        out_type=jax.ShapeDtypeStruct((E, D_IN), jnp.float32),
        scratch_types=[
            pltpu.VMEM((EPW,), jnp.int32),
            pltpu.VMEM((CHUNK, D_IN), jnp.float32),
            pltpu.VMEM((TAIL,), jnp.int32),
            pltpu.VMEM((TAIL, D_IN), jnp.float32),
            pltpu.SemaphoreType.DMA,
        ])
    def k(table_hbm, idx_hbm, out_hbm, idx_all, buf, idx_t, buf_t, sem):
        wid = lax.axis_index("s") * NC + lax.axis_index("c")
        base = wid * EPW
        pltpu.sync_copy(idx_hbm.at[pl.ds(base, EPW)], idx_all)

        def body(j, _):
            off = j * CHUNK
            pltpu.async_copy(
                table_hbm.at[idx_all.at[pl.ds(off, CHUNK)]], buf, sem).wait()
            pltpu.sync_copy(buf, out_hbm.at[pl.ds(base + off, CHUNK)])
            return 0

        lax.fori_loop(0, NFULL, body, 0)
        pltpu.sync_copy(idx_hbm.at[pl.ds(base + NFULL * CHUNK, TAIL)], idx_t)
        pltpu.async_copy(table_hbm.at[idx_t], buf_t, sem).wait()
        pltpu.sync_copy(buf_t, out_hbm.at[pl.ds(base + NFULL * CHUNK, TAIL)])

    return k(table, idx)


def _sc_scatter(msg, dst, zeros, width):
    mesh = plsc.VectorSubcoreMesh(core_axis_name="c", subcore_axis_name="s")

    @functools.partial(
        pl.kernel, mesh=mesh,
        compiler_params=pltpu.CompilerParams(use_tc_tiling_on_sc=False),
        out_type=jax.ShapeDtypeStruct((2 * N, width), jnp.float32),
        scratch_types=[
            pltpu.VMEM_SHARED((N, width), jnp.float32),
            pltpu.VMEM((CHUNK,), jnp.int32),
            pltpu.VMEM((CHUNK, width), jnp.float32),
            pltpu.VMEM((TAIL,), jnp.int32),
            pltpu.VMEM((TAIL, width), jnp.float32),
        ])
    def k(msg_hbm, dst_hbm, zeros_hbm, out_hbm, acc, idx_v, rows_v, idx_t, rows_t):
        cid = lax.axis_index("c")
        sid = lax.axis_index("s")
        wid = sid * NC + cid
        base = wid * EPW
        pltpu.sync_copy(zeros_hbm.at[pl.ds(sid * NPT, NPT)],
                        acc.at[pl.ds(sid * NPT, NPT)])
        plsc.subcore_barrier()

        def body(j, _):
            off = base + j * CHUNK
            pltpu.sync_copy(dst_hbm.at[pl.ds(off, CHUNK)], idx_v)
            pltpu.sync_copy(msg_hbm.at[pl.ds(off, CHUNK)], rows_v)
            pltpu.sync_copy(rows_v, acc.at[idx_v], add=True)
            return 0

        lax.fori_loop(0, NFULL, body, 0)
        off_t = base + NFULL * CHUNK
        pltpu.sync_copy(dst_hbm.at[pl.ds(off_t, TAIL)], idx_t)
        pltpu.sync_copy(msg_hbm.at[pl.ds(off_t, TAIL)], rows_t)
        pltpu.sync_copy(rows_t, acc.at[idx_t], add=True)
        plsc.subcore_barrier()
        pltpu.sync_copy(acc.at[pl.ds(sid * NPT, NPT)],
                        out_hbm.at[pl.ds(cid * N + sid * NPT, NPT)])

    return k(msg, dst, zeros)


def _gather_rows(table, idx):
    return jnp.take(table, idx, axis=0)


def _scatter_sum(rows, dst, width):
    s = jax.ops.segment_sum(rows, dst, num_segments=N)
    return jnp.stack([s, jnp.zeros_like(s)], axis=0)


def kernel(x, edge_index, edge_attr, A1, b1, A2, b2, root1, bias1, root2, bias2, Wo, bo):
    src = edge_index[0]
    dst = edge_index[1]
    A2m = A2.reshape(HID, D_IN, HID).reshape(HID * D_IN, HID)
    B2 = b2.reshape(D_IN, HID)
    R = jnp.repeat(jnp.eye(HID, dtype=jnp.float32), D_IN, axis=1)
    T = jnp.tile(jnp.eye(D_IN, dtype=jnp.float32), (1, HID))
    b1r = b1.reshape(1, HID)
    bias1r = bias1.reshape(1, HID)
    bias2r = bias2.reshape(1, HID)
    bor = bo.reshape(1, 1)

    zeros48 = jnp.zeros((N, HID + 16), jnp.float32)
    zeros32 = jnp.zeros((N, HID), jnp.float32)

    xj = _sc_gather(x, src)
    msg1 = _edge_messages(edge_attr, xj, A1, b1r, A2m, B2, R, T, True)
    part1 = _sc_scatter(msg1, dst, zeros48, HID + 16).reshape(2, N, HID + 16)
    h, cnt = _node1(part1, x, root1, bias1r)

    hj = _sc_gather(h, src)
    msg2 = _edge_messages(edge_attr, hj, A1, b1r, A2m, B2, R, T, False)
    part2 = _sc_scatter(msg2, dst, zeros32, HID).reshape(2, N, HID)
    out = _node2(part2, cnt, h, root2, bias2r, Wo, bor)
    return out


# speedup vs baseline: 1.1104x; 1.1104x over previous
import functools

import jax
import jax.numpy as jnp
from jax import lax
from jax.experimental import pallas as pl
from jax.experimental.pallas import tpu as pltpu
from jax.experimental.pallas import tpu_sc as plsc

N = 10000
E = 160000
D_IN = 32
D_EDGE = 16
HID = 32

ET = 640
EGRID = E // ET

NC = 2
NS = 16
NW = NC * NS
EPW = E // NW
CHUNK = 128
NFULL = EPW // CHUNK
TAIL = EPW - NFULL * CHUNK
NPT = N // NS


def _edges_body(ea_ref, xj_ref, A1_ref, b1_ref, A2m_ref, B2_ref, R_ref, out_ref, *, ones_col):
    ea = ea_ref[...]
    xj = xj_ref[...]
    h = jnp.maximum(
        jnp.dot(ea, A1_ref[...], preferred_element_type=jnp.float32) + b1_ref[...],
        0.0)
    h_rep = jnp.dot(h, R_ref[...], preferred_element_type=jnp.float32)
    x_rep = jnp.tile(xj, (1, HID))
    z = h_rep * x_rep
    msg = jnp.dot(z, A2m_ref[...], preferred_element_type=jnp.float32)
    msg = msg + jnp.dot(xj, B2_ref[...], preferred_element_type=jnp.float32)
    if ones_col:
        pad = jnp.concatenate(
            [msg, jnp.ones((ET, 1), jnp.float32), jnp.zeros((ET, 15), jnp.float32)],
            axis=1)
        out_ref[...] = pad
    else:
        out_ref[...] = msg


def _edge_messages(edge_attr, xj, A1, b1, A2m, B2, R, ones_col):
    width = HID + 16 if ones_col else HID
    return pl.pallas_call(
        functools.partial(_edges_body, ones_col=ones_col),
        grid=(EGRID,),
        in_specs=[
            pl.BlockSpec((ET, D_EDGE), lambda i: (i, 0)),
            pl.BlockSpec((ET, D_IN), lambda i: (i, 0)),
            pl.BlockSpec((D_EDGE, HID), lambda i: (0, 0)),
            pl.BlockSpec((1, HID), lambda i: (0, 0)),
            pl.BlockSpec((HID * D_IN, HID), lambda i: (0, 0)),
            pl.BlockSpec((D_IN, HID), lambda i: (0, 0)),
            pl.BlockSpec((HID, HID * D_IN), lambda i: (0, 0)),
        ],
        out_specs=pl.BlockSpec((ET, width), lambda i: (i, 0)),
        out_shape=jax.ShapeDtypeStruct((E, width), jnp.float32),
    )(edge_attr, xj, A1, b1, A2m, B2, R)


def _node1_body(part_ref, x_ref, root_ref, bias_ref, h_ref, cnt_ref):
    p = part_ref[0] + part_ref[1]
    cnt = p[:, HID:HID + 1]
    denom = jnp.maximum(cnt, 1.0)
    agg = p[:, :HID] / denom
    r = jnp.dot(x_ref[...], root_ref[...], preferred_element_type=jnp.float32)
    h_ref[...] = jnp.maximum(agg + r + bias_ref[...], 0.0)
    cnt_ref[...] = cnt


def _node2_body(part_ref, cnt_ref, h_ref, root_ref, bias_ref, Wo_ref, bo_ref, out_ref):
    p = part_ref[0] + part_ref[1]
    denom = jnp.maximum(cnt_ref[...], 1.0)
    agg = p / denom
    r = jnp.dot(h_ref[...], root_ref[...], preferred_element_type=jnp.float32)
    h2 = jnp.maximum(agg + r + bias_ref[...], 0.0)
    logit = jnp.dot(h2, Wo_ref[...], preferred_element_type=jnp.float32) + bo_ref[...]
    out_ref[...] = jax.nn.sigmoid(logit)


def _node1(partial, x, root1, bias1):
    return pl.pallas_call(
        _node1_body,
        out_shape=(jax.ShapeDtypeStruct((N, HID), jnp.float32),
                   jax.ShapeDtypeStruct((N, 1), jnp.float32)),
    )(partial, x, root1, bias1)


def _node2(partial, cnt, h, root2, bias2, Wo, bo):
    return pl.pallas_call(
        _node2_body,
        out_shape=jax.ShapeDtypeStruct((N, 1), jnp.float32),
    )(partial, cnt, h, root2, bias2, Wo, bo)


def _sc_gather(table, idx):
    mesh = plsc.VectorSubcoreMesh(core_axis_name="c", subcore_axis_name="s")

    @functools.partial(
        pl.kernel, mesh=mesh,
        compiler_params=pltpu.CompilerParams(use_tc_tiling_on_sc=False),
        out_type=jax.ShapeDtypeStruct((E, D_IN), jnp.float32),
        scratch_types=[
            pltpu.VMEM((EPW,), jnp.int32),
            pltpu.VMEM((CHUNK, D_IN), jnp.float32),
            pltpu.VMEM((TAIL,), jnp.int32),
            pltpu.VMEM((TAIL, D_IN), jnp.float32),
            pltpu.SemaphoreType.DMA,
        ])
    def k(table_hbm, idx_hbm, out_hbm, idx_all, buf, idx_t, buf_t, sem):
        wid = lax.axis_index("s") * NC + lax.axis_index("c")
        base = wid * EPW
        pltpu.sync_copy(idx_hbm.at[pl.ds(base, EPW)], idx_all)

        def body(j, _):
            off = j * CHUNK
            pltpu.async_copy(
                table_hbm.at[idx_all.at[pl.ds(off, CHUNK)]], buf, sem).wait()
            pltpu.sync_copy(buf, out_hbm.at[pl.ds(base + off, CHUNK)])
            return 0

        lax.fori_loop(0, NFULL, body, 0)
        pltpu.sync_copy(idx_hbm.at[pl.ds(base + NFULL * CHUNK, TAIL)], idx_t)
        pltpu.async_copy(table_hbm.at[idx_t], buf_t, sem).wait()
        pltpu.sync_copy(buf_t, out_hbm.at[pl.ds(base + NFULL * CHUNK, TAIL)])

    return k(table, idx)


def _sc_scatter(msg, dst, zeros, width):
    mesh = plsc.VectorSubcoreMesh(core_axis_name="c", subcore_axis_name="s")

    @functools.partial(
        pl.kernel, mesh=mesh,
        compiler_params=pltpu.CompilerParams(use_tc_tiling_on_sc=False),
        out_type=jax.ShapeDtypeStruct((2 * N, width), jnp.float32),
        scratch_types=[
            pltpu.VMEM_SHARED((N, width), jnp.float32),
            pltpu.VMEM((CHUNK,), jnp.int32),
            pltpu.VMEM((CHUNK, width), jnp.float32),
            pltpu.VMEM((TAIL,), jnp.int32),
            pltpu.VMEM((TAIL, width), jnp.float32),
        ])
    def k(msg_hbm, dst_hbm, zeros_hbm, out_hbm, acc, idx_v, rows_v, idx_t, rows_t):
        cid = lax.axis_index("c")
        sid = lax.axis_index("s")
        wid = sid * NC + cid
        base = wid * EPW
        pltpu.sync_copy(zeros_hbm.at[pl.ds(sid * NPT, NPT)],
                        acc.at[pl.ds(sid * NPT, NPT)])
        plsc.subcore_barrier()

        def body(j, _):
            off = base + j * CHUNK
            pltpu.sync_copy(dst_hbm.at[pl.ds(off, CHUNK)], idx_v)
            pltpu.sync_copy(msg_hbm.at[pl.ds(off, CHUNK)], rows_v)
            pltpu.sync_copy(rows_v, acc.at[idx_v], add=True)
            return 0

        lax.fori_loop(0, NFULL, body, 0)
        off_t = base + NFULL * CHUNK
        pltpu.sync_copy(dst_hbm.at[pl.ds(off_t, TAIL)], idx_t)
        pltpu.sync_copy(msg_hbm.at[pl.ds(off_t, TAIL)], rows_t)
        pltpu.sync_copy(rows_t, acc.at[idx_t], add=True)
        plsc.subcore_barrier()
        pltpu.sync_copy(acc.at[pl.ds(sid * NPT, NPT)],
                        out_hbm.at[pl.ds(cid * N + sid * NPT, NPT)])

    return k(msg, dst, zeros)


def _gather_rows(table, idx):
    return jnp.take(table, idx, axis=0)


def _scatter_sum(rows, dst, width):
    s = jax.ops.segment_sum(rows, dst, num_segments=N)
    return jnp.stack([s, jnp.zeros_like(s)], axis=0)


def kernel(x, edge_index, edge_attr, A1, b1, A2, b2, root1, bias1, root2, bias2, Wo, bo):
    src = edge_index[0]
    dst = edge_index[1]
    A2m = A2.reshape(HID, D_IN, HID).reshape(HID * D_IN, HID)
    B2 = b2.reshape(D_IN, HID)
    R = jnp.repeat(jnp.eye(HID, dtype=jnp.float32), D_IN, axis=1)
    b1r = b1.reshape(1, HID)
    bias1r = bias1.reshape(1, HID)
    bias2r = bias2.reshape(1, HID)
    bor = bo.reshape(1, 1)

    zeros48 = jnp.zeros((N, HID + 16), jnp.float32)
    zeros32 = jnp.zeros((N, HID), jnp.float32)

    xj = _sc_gather(x, src)
    msg1 = _edge_messages(edge_attr, xj, A1, b1r, A2m, B2, R, True)
    part1 = _sc_scatter(msg1, dst, zeros48, HID + 16).reshape(2, N, HID + 16)
    h, cnt = _node1(part1, x, root1, bias1r)

    hj = _sc_gather(h, src)
    msg2 = _edge_messages(edge_attr, hj, A1, b1r, A2m, B2, R, False)
    part2 = _sc_scatter(msg2, dst, zeros32, HID).reshape(2, N, HID)
    out = _node2(part2, cnt, h, root2, bias2r, Wo, bor)
    return out
